# baseline (speedup 1.0000x reference)
import functools

import jax
import jax.numpy as jnp
from jax import lax
from jax.experimental import pallas as pl
from jax.experimental.pallas import tpu as pltpu
from jax.experimental.pallas import tpu_sc as plsc

DIM_ = 64
ROWS_ = 800
N_WORDS_ = 80
LANES_ = 16
SPW_ = 0.1 / (DIM_ - 1)
SCALE_ = 0.5 / N_WORDS_


def _word_meta(g):
    if g < 32:
        return (g // 4) * 100 + g % 4, g // 4, 0
    if g < 64:
        h = g - 32
        return (h // 4) * 100 + 10 + h % 4, h // 4, 1
    h = g - 64
    return (h // 2) * 100 + 20 + h % 2, h // 2, 2


def _body(table, out, slab_v, tv_v, sem):
    pltpu.async_copy(table.at[pl.ds(0, ROWS_)], slab_v, sem).wait()

    lanes = lax.iota(jnp.int32, 16)
    fzero = jnp.zeros((16,), jnp.float32)
    fone = jnp.ones((16,), jnp.float32)

    acc = fzero
    f = fzero
    for g in range(N_WORDS_):
        w, d, cls = _word_meta(g)
        onehot = jnp.where(lanes == d, fone, fzero)
        s0 = slab_v[w, 0:16]
        a0 = jnp.abs(s0)
        if cls == 0:
            fg = jnp.where(s0 <= 0.0, a0 + 0.1, -0.1 * s0) - SPW_ * a0
        elif cls == 1:
            fg = jnp.where(s0 >= 0.0, a0 + 0.1, 0.1 * s0) - SPW_ * a0
        else:
            fg = 2.0 * a0
        f = f + fg * onehot
        if cls != 2:
            acc = acc + a0
            for k in range(1, 4):
                acc = acc + jnp.abs(slab_v[w, 16 * k:16 * (k + 1)])

    total_vec = f + SPW_ * acc
    total = jnp.float32(0.0)
    for j in range(16):
        total = total + total_vec[j]
    total = total * SCALE_
    tv_v[...] = jnp.full((16,), total, jnp.float32)
    pltpu.sync_copy(tv_v.at[0:1], out)


_sc_call = functools.partial(
    pl.kernel,
    mesh=plsc.VectorSubcoreMesh(core_axis_name="c", subcore_axis_name="s",
                                num_cores=1, num_subcores=1),
    out_type=jax.ShapeDtypeStruct((1,), jnp.float32),
    compiler_params=pltpu.CompilerParams(use_tc_tiling_on_sc=True),
    scratch_types=[
        pltpu.VMEM((ROWS_, DIM_), jnp.float32),
        pltpu.VMEM((LANES_,), jnp.float32),
        pltpu.SemaphoreType.DMA,
    ],
)(_body)


@jax.jit
def kernel(embeddings):
    out = _sc_call(embeddings)
    return jnp.reshape(out, ())


# speedup vs baseline: 2.6368x; 2.6368x over previous
import functools

import jax
import jax.numpy as jnp
from jax import lax
from jax.experimental import pallas as pl
from jax.experimental.pallas import tpu as pltpu
from jax.experimental.pallas import tpu_sc as plsc

DIM_ = 64
ROWS_ = 800
N_WORDS_ = 80
LANES_ = 16
SPW_ = 0.1 / (DIM_ - 1)
SCALE_ = 0.5 / N_WORDS_


def _ids_for(g):
    d = jnp.where(g < 32, g >> 2,
                  jnp.where(g < 64, (g - 32) >> 2, (g - 64) >> 1))
    off = jnp.where(g < 32, g & 3,
                    jnp.where(g < 64, 10 + ((g - 32) & 3),
                              20 + ((g - 64) & 1)))
    return d, d * 100 + off


def _body(table, out, idx_v, rows_v, tv_v, sem):
    lanes = lax.iota(jnp.int32, 16)
    for t in range(5):
        _, word = _ids_for(t * 16 + lanes)
        idx_v[16 * t:16 * (t + 1)] = word
    gather = pltpu.async_copy(table.at[idx_v], rows_v, sem)

    fzero = jnp.zeros((16,), jnp.float32)
    fone = jnp.ones((16,), jnp.float32)
    gather.wait()

    def block(b, carry):
        f, acc = carry
        base = b * 16
        d, _ = _ids_for(base + lanes)
        for j in range(16):
            gj = base + j
            s0 = rows_v[gj, 0:16]
            a0 = jnp.abs(s0)
            fp = jnp.where(s0 <= 0.0, a0 + 0.1, -0.1 * s0) - SPW_ * a0
            fn = jnp.where(s0 >= 0.0, a0 + 0.1, 0.1 * s0) - SPW_ * a0
            fu = 2.0 * a0
            wp = jnp.where(gj < 32, jnp.float32(1.0), jnp.float32(0.0))
            wn = jnp.where(jnp.logical_and(gj >= 32, gj < 64),
                           jnp.float32(1.0), jnp.float32(0.0))
            wu = jnp.where(gj >= 64, jnp.float32(1.0), jnp.float32(0.0))
            onehot = jnp.where(lanes == d[j], fone, fzero)
            f = f + (fp * wp + fn * wn + fu * wu) * onehot
            wsp = jnp.where(gj < 64, jnp.float32(1.0), jnp.float32(0.0))
            blk = a0
            for k in range(1, 4):
                blk = blk + jnp.abs(rows_v[gj, 16 * k:16 * (k + 1)])
            acc = acc + wsp * blk
        return f, acc

    f, acc = lax.fori_loop(0, 5, block, (fzero, fzero))

    total_vec = f + SPW_ * acc
    total = jnp.float32(0.0)
    for j in range(16):
        total = total + total_vec[j]
    total = total * SCALE_
    tv_v[...] = jnp.full((16,), total, jnp.float32)
    pltpu.sync_copy(tv_v.at[0:1], out)


_sc_call = functools.partial(
    pl.kernel,
    mesh=plsc.VectorSubcoreMesh(core_axis_name="c", subcore_axis_name="s",
                                num_cores=1, num_subcores=1),
    out_type=jax.ShapeDtypeStruct((1,), jnp.float32),
    compiler_params=pltpu.CompilerParams(use_tc_tiling_on_sc=False),
    scratch_types=[
        pltpu.VMEM((N_WORDS_,), jnp.int32),
        pltpu.VMEM((N_WORDS_, DIM_), jnp.float32),
        pltpu.VMEM((LANES_,), jnp.float32),
        pltpu.SemaphoreType.DMA,
    ],
)(_body)


@jax.jit
def kernel(embeddings):
    out = _sc_call(embeddings[:ROWS_])
    return jnp.reshape(out, ())


# speedup vs baseline: 2.8414x; 1.0776x over previous
import functools

import jax
import jax.numpy as jnp
from jax import lax
from jax.experimental import pallas as pl
from jax.experimental.pallas import tpu as pltpu
from jax.experimental.pallas import tpu_sc as plsc

DIM_ = 64
ROWS_ = 800
N_WORDS_ = 80
LANES_ = 16
SPW_ = 0.1 / (DIM_ - 1)
SCALE_ = 0.5 / N_WORDS_


def _body(table, out, idx_v, rows_v, tv_v, sem):
    lanes = lax.iota(jnp.int32, 16)
    for t in range(5):
        g = t * 16 + lanes
        d = jnp.where(g < 32, g >> 2,
                      jnp.where(g < 64, (g - 32) >> 2, (g - 64) >> 1))
        off = jnp.where(g < 32, g & 3,
                        jnp.where(g < 64, 10 + ((g - 32) & 3),
                                  20 + ((g - 64) & 1)))
        idx_v[16 * t:16 * (t + 1)] = d * 100 + off
    gather = pltpu.async_copy(table.at[idx_v], rows_v, sem)

    fzero = jnp.zeros((16,), jnp.float32)
    fone = jnp.ones((16,), jnp.float32)
    onehots = [jnp.where(lanes == d, fone, fzero) for d in range(8)]
    gather.wait()

    f = fzero
    accs = [fzero] * 4
    for d in range(8):
        fp_sum = fzero
        for j in range(4):
            s0 = rows_v[4 * d + j, 0:16]
            a0 = jnp.abs(s0)
            fp_sum = fp_sum + (jnp.where(s0 <= 0.0, a0 + 0.1, -0.1 * s0)
                               - SPW_ * a0)
            accs[0] = accs[0] + a0
            for k in range(1, 4):
                accs[k] = accs[k] + jnp.abs(
                    rows_v[4 * d + j, 16 * k:16 * (k + 1)])
        fn_sum = fzero
        for j in range(4):
            s0 = rows_v[32 + 4 * d + j, 0:16]
            a0 = jnp.abs(s0)
            fn_sum = fn_sum + (jnp.where(s0 >= 0.0, a0 + 0.1, 0.1 * s0)
                               - SPW_ * a0)
            accs[0] = accs[0] + a0
            for k in range(1, 4):
                accs[k] = accs[k] + jnp.abs(
                    rows_v[32 + 4 * d + j, 16 * k:16 * (k + 1)])
        fu_sum = fzero
        for j in range(2):
            s0 = rows_v[64 + 2 * d + j, 0:16]
            fu_sum = fu_sum + jnp.abs(s0)
        f = f + (fp_sum + fn_sum + 2.0 * fu_sum) * onehots[d]

    total_vec = f + SPW_ * ((accs[0] + accs[1]) + (accs[2] + accs[3]))
    total = jnp.float32(0.0)
    for j in range(16):
        total = total + total_vec[j]
    total = total * SCALE_
    tv_v[...] = jnp.full((16,), total, jnp.float32)
    pltpu.sync_copy(tv_v.at[0:1], out)


_sc_call = functools.partial(
    pl.kernel,
    mesh=plsc.VectorSubcoreMesh(core_axis_name="c", subcore_axis_name="s",
                                num_cores=1, num_subcores=1),
    out_type=jax.ShapeDtypeStruct((1,), jnp.float32),
    compiler_params=pltpu.CompilerParams(use_tc_tiling_on_sc=False),
    scratch_types=[
        pltpu.VMEM((N_WORDS_,), jnp.int32),
        pltpu.VMEM((N_WORDS_, DIM_), jnp.float32),
        pltpu.VMEM((LANES_,), jnp.float32),
        pltpu.SemaphoreType.DMA,
    ],
)(_body)


@jax.jit
def kernel(embeddings):
    out = _sc_call(embeddings[:ROWS_])
    return jnp.reshape(out, ())


# speedup vs baseline: 2.8436x; 1.0008x over previous
import functools

import jax
import jax.numpy as jnp
from jax import lax
from jax.experimental import pallas as pl
from jax.experimental.pallas import tpu as pltpu
from jax.experimental.pallas import tpu_sc as plsc

DIM_ = 64
ROWS_ = 800
N_WORDS_ = 80
LANES_ = 16
SPW_ = 0.1 / (DIM_ - 1)
SCALE_ = 0.5 / N_WORDS_


def _body(table, out, idx_v, rows_v, tv_v, sem):
    lanes = lax.iota(jnp.int32, 16)
    for t in range(5):
        g = t * 16 + lanes
        d = jnp.where(g < 32, g >> 2,
                      jnp.where(g < 64, (g - 32) >> 2, (g - 64) >> 1))
        off = jnp.where(g < 32, g & 3,
                        jnp.where(g < 64, 10 + ((g - 32) & 3),
                                  20 + ((g - 64) & 1)))
        idx_v[16 * t:16 * (t + 1)] = d * 100 + off
    gather = pltpu.async_copy(table.at[idx_v], rows_v, sem)

    fzero = jnp.zeros((16,), jnp.float32)
    fone = jnp.ones((16,), jnp.float32)
    onehots = [jnp.where(lanes == d, fone, fzero) for d in range(8)]
    gather.wait()

    f = fzero
    accs = [fzero] * 4
    for d in range(8):
        fp_sum = fzero
        for j in range(4):
            s0 = rows_v[4 * d + j, 0:16]
            a0 = jnp.abs(s0)
            fp_sum = fp_sum + (jnp.where(s0 <= 0.0, a0 + 0.1, -0.1 * s0)
                               - SPW_ * a0)
            accs[0] = accs[0] + a0
            for k in range(1, 4):
                accs[k] = accs[k] + jnp.abs(
                    rows_v[4 * d + j, 16 * k:16 * (k + 1)])
        fn_sum = fzero
        for j in range(4):
            s0 = rows_v[32 + 4 * d + j, 0:16]
            a0 = jnp.abs(s0)
            fn_sum = fn_sum + (jnp.where(s0 >= 0.0, a0 + 0.1, 0.1 * s0)
                               - SPW_ * a0)
            accs[0] = accs[0] + a0
            for k in range(1, 4):
                accs[k] = accs[k] + jnp.abs(
                    rows_v[32 + 4 * d + j, 16 * k:16 * (k + 1)])
        fu_sum = fzero
        for j in range(2):
            s0 = rows_v[64 + 2 * d + j, 0:16]
            fu_sum = fu_sum + jnp.abs(s0)
        f = f + (fp_sum + fn_sum + 2.0 * fu_sum) * onehots[d]

    total_vec = f + SPW_ * ((accs[0] + accs[1]) + (accs[2] + accs[3]))
    total = jnp.float32(0.0)
    for j in range(16):
        total = total + total_vec[j]
    total = total * SCALE_
    tv_v[...] = jnp.full((16,), total, jnp.float32)
    pltpu.sync_copy(tv_v.at[0:1], out)


_sc_call = functools.partial(
    pl.kernel,
    mesh=plsc.VectorSubcoreMesh(core_axis_name="c", subcore_axis_name="s",
                                num_cores=1, num_subcores=1),
    out_type=jax.ShapeDtypeStruct((1,), jnp.float32),
    compiler_params=pltpu.CompilerParams(use_tc_tiling_on_sc=False,
                                         skip_device_barrier=True),
    scratch_types=[
        pltpu.VMEM((N_WORDS_,), jnp.int32),
        pltpu.VMEM((N_WORDS_, DIM_), jnp.float32),
        pltpu.VMEM((LANES_,), jnp.float32),
        pltpu.SemaphoreType.DMA,
    ],
)(_body)


@jax.jit
def kernel(embeddings):
    out = _sc_call(embeddings[:ROWS_])
    return jnp.reshape(out, ())


# speedup vs baseline: 2.8946x; 1.0180x over previous
import functools

import jax
import jax.numpy as jnp
from jax import lax
from jax.experimental import pallas as pl
from jax.experimental.pallas import tpu as pltpu
from jax.experimental.pallas import tpu_sc as plsc

DIM_ = 64
ROWS_ = 800
N_WORDS_ = 80
LANES_ = 16
SPW_ = 0.1 / (DIM_ - 1)
SCALE_ = 0.5 / N_WORDS_


def _body(table, out, idx_v, rows_v, tv_v, sem):
    lanes = lax.iota(jnp.int32, 16)
    for t in range(5):
        g = t * 16 + lanes
        d = jnp.where(g < 32, g >> 2,
                      jnp.where(g < 64, (g - 32) >> 2, (g - 64) >> 1))
        off = jnp.where(g < 32, g & 3,
                        jnp.where(g < 64, 10 + ((g - 32) & 3),
                                  20 + ((g - 64) & 1)))
        idx_v[16 * t:16 * (t + 1)] = d * 100 + off
    gather = pltpu.async_copy(table.at[idx_v], rows_v, sem)

    fzero = jnp.zeros((16,), jnp.float32)
    fone = jnp.ones((16,), jnp.float32)
    gather.wait()

    def dim_block(d, carry):
        f, acc0, acc1, acc2, acc3 = carry
        onehot = jnp.where(lanes == d, fone, fzero)
        fp_sum = fzero
        for j in range(4):
            r = 4 * d + j
            s0 = rows_v[r, 0:16]
            a0 = jnp.abs(s0)
            fp_sum = fp_sum + (jnp.where(s0 <= 0.0, a0 + 0.1, -0.1 * s0)
                               - SPW_ * a0)
            acc0 = acc0 + a0
            acc1 = acc1 + jnp.abs(rows_v[r, 16:32])
            acc2 = acc2 + jnp.abs(rows_v[r, 32:48])
            acc3 = acc3 + jnp.abs(rows_v[r, 48:64])
        fn_sum = fzero
        for j in range(4):
            r = 32 + 4 * d + j
            s0 = rows_v[r, 0:16]
            a0 = jnp.abs(s0)
            fn_sum = fn_sum + (jnp.where(s0 >= 0.0, a0 + 0.1, 0.1 * s0)
                               - SPW_ * a0)
            acc0 = acc0 + a0
            acc1 = acc1 + jnp.abs(rows_v[r, 16:32])
            acc2 = acc2 + jnp.abs(rows_v[r, 32:48])
            acc3 = acc3 + jnp.abs(rows_v[r, 48:64])
        fu_sum = (jnp.abs(rows_v[64 + 2 * d, 0:16])
                  + jnp.abs(rows_v[65 + 2 * d, 0:16]))
        f = f + (fp_sum + fn_sum + 2.0 * fu_sum) * onehot
        return f, acc0, acc1, acc2, acc3

    f, acc0, acc1, acc2, acc3 = lax.fori_loop(
        0, 8, dim_block, (fzero, fzero, fzero, fzero, fzero))

    total_vec = f + SPW_ * ((acc0 + acc1) + (acc2 + acc3))
    total = jnp.float32(0.0)
    for j in range(16):
        total = total + total_vec[j]
    total = total * SCALE_
    tv_v[...] = jnp.full((16,), total, jnp.float32)
    pltpu.sync_copy(tv_v.at[0:1], out)


_sc_call = functools.partial(
    pl.kernel,
    mesh=plsc.VectorSubcoreMesh(core_axis_name="c", subcore_axis_name="s",
                                num_cores=1, num_subcores=1),
    out_type=jax.ShapeDtypeStruct((1,), jnp.float32),
    compiler_params=pltpu.CompilerParams(use_tc_tiling_on_sc=False),
    scratch_types=[
        pltpu.VMEM((N_WORDS_,), jnp.int32),
        pltpu.VMEM((N_WORDS_, DIM_), jnp.float32),
        pltpu.VMEM((LANES_,), jnp.float32),
        pltpu.SemaphoreType.DMA,
    ],
)(_body)


@jax.jit
def kernel(embeddings):
    out = _sc_call(embeddings[:ROWS_])
    return jnp.reshape(out, ())
